# Initial kernel scaffold; baseline (speedup 1.0000x reference)
import jax
import jax.numpy as jnp
from jax.experimental import pallas as pl


def kernel(x, edge_index, W1, b1, W2, b2, W3, b3, Wc, bc):
    raise NotImplementedError("write your pallas kernel here")


# speedup vs baseline: 10.0862x; 10.0862x over previous
import functools

import jax
import jax.numpy as jnp
from jax import lax
from jax.experimental import pallas as pl
from jax.experimental.pallas import tpu as pltpu
from jax.experimental.pallas import tpu_sc as plsc

N, D, H, C, E = 10000, 128, 128, 8, 320000
NP = 10240
NC, NS, VL = 2, 16, 16
CK = 128
CH = -(-E // (NC * NS * CK))
EPAD = NC * NS * CK * CH
SL = NP // NS
WD = 16
RB = 512
G = NP // RB

_mesh = plsc.VectorSubcoreMesh(core_axis_name="c", subcore_axis_name="s")


@functools.partial(
    pl.kernel, mesh=_mesh,
    out_type=jax.ShapeDtypeStruct((NC, NP, WD), jnp.float32),
    scratch_types=[
        pltpu.VMEM((CK,), jnp.int32),
        pltpu.VMEM((CK, WD), jnp.float32),
        pltpu.VMEM((VL, WD), jnp.float32),
        pltpu.VMEM_SHARED((NP, WD), jnp.float32),
    ])
def _deg_kernel(dst_hbm, out_hbm, dst_v, ones_v, zb, degsh):
    c = lax.axis_index("c")
    s = lax.axis_index("s")
    one = jnp.ones((VL,), jnp.float32)
    zero = jnp.zeros((VL,), jnp.float32)
    for i in range(CK):
        ones_v[i, pl.ds(0, VL)] = one
    for i in range(VL):
        zb[i, pl.ds(0, VL)] = zero
    for k in range(SL // VL):
        pltpu.sync_copy(zb, degsh.at[pl.ds(s * SL + k * VL, VL)])
    plsc.subcore_barrier()
    wid = c * NS + s

    def body(j, carry):
        base = (wid * CH + j) * CK
        pltpu.sync_copy(dst_hbm.at[pl.ds(base, CK)], dst_v)
        pltpu.sync_copy(ones_v, degsh.at[dst_v], add=True)
        return carry

    lax.fori_loop(0, CH, body, 0)
    plsc.subcore_barrier()
    pltpu.sync_copy(degsh.at[pl.ds(s * SL, SL)], out_hbm.at[c, pl.ds(s * SL, SL)])


@functools.partial(
    pl.kernel, mesh=_mesh,
    out_type=jax.ShapeDtypeStruct((NC, NP, D), jnp.float32),
    scratch_types=[
        pltpu.VMEM((CK,), jnp.int32),
        pltpu.VMEM((CK,), jnp.int32),
        pltpu.VMEM((CK, D), jnp.float32),
        pltpu.VMEM((VL, D), jnp.float32),
        pltpu.VMEM_SHARED((NP, D), jnp.float32),
        pltpu.SemaphoreType.DMA,
    ])
def _agg_kernel(hs_hbm, src_hbm, dst_hbm, out_hbm, src_v, dst_v, rows_v, zb, aggsh, sem):
    c = lax.axis_index("c")
    s = lax.axis_index("s")
    zero = jnp.zeros((VL,), jnp.float32)
    for i in range(VL):
        for j in range(D // VL):
            zb[i, pl.ds(j * VL, VL)] = zero
    for k in range(SL // VL):
        pltpu.sync_copy(zb, aggsh.at[pl.ds(s * SL + k * VL, VL)])
    plsc.subcore_barrier()
    wid = c * NS + s

    def body(j, carry):
        base = (wid * CH + j) * CK
        pltpu.sync_copy(src_hbm.at[pl.ds(base, CK)], src_v)
        pltpu.sync_copy(dst_hbm.at[pl.ds(base, CK)], dst_v)
        pltpu.async_copy(hs_hbm.at[src_v], rows_v, sem).wait()
        pltpu.sync_copy(rows_v, aggsh.at[dst_v], add=True)
        return carry

    lax.fori_loop(0, CH, body, 0)
    plsc.subcore_barrier()
    pltpu.sync_copy(aggsh.at[pl.ds(s * SL, SL)], out_hbm.at[c, pl.ds(s * SL, SL)])


def _row_ids(i):
    return i * RB + lax.broadcasted_iota(jnp.int32, (RB, 1), 0)


def _t_first_body(x_ref, w_ref, dinv_ref, o_ref):
    i = pl.program_id(0)
    h = lax.dot_general(x_ref[...], w_ref[...], (((1,), (1,)), ((), ())),
                        preferred_element_type=jnp.float32)
    o_ref[...] = jnp.where(_row_ids(i) < N, h * dinv_ref[...], 0.0)


def _t_mid_body(agg_ref, hs_ref, dinv_ref, b_ref, w_ref, o_ref):
    i = pl.program_id(0)
    x = jnp.maximum(dinv_ref[...] * (agg_ref[0] + agg_ref[1] + hs_ref[...])
                    + b_ref[...], 0.0)
    h = lax.dot_general(x, w_ref[...], (((1,), (1,)), ((), ())),
                        preferred_element_type=jnp.float32)
    o_ref[...] = jnp.where(_row_ids(i) < N, h * dinv_ref[...], 0.0)


def _t_final_body(agg_ref, hs_ref, dinv_ref, b_ref, wc_ref, bc_ref, o_ref, acc_ref):
    i = pl.program_id(0)

    @pl.when(i == 0)
    def _():
        acc_ref[...] = jnp.zeros_like(acc_ref)

    x = jnp.maximum(dinv_ref[...] * (agg_ref[0] + agg_ref[1] + hs_ref[...])
                    + b_ref[...], 0.0)
    x = jnp.where(_row_ids(i) < N, x, 0.0)
    acc_ref[...] += jnp.sum(x, axis=0, keepdims=True)

    @pl.when(i == G - 1)
    def _():
        g = acc_ref[...] * (1.0 / N)
        z = lax.dot_general(g, wc_ref[...], (((1,), (1,)), ((), ())),
                            preferred_element_type=jnp.float32) + bc_ref[...]
        o_ref[...] = 1.0 / (1.0 + jnp.exp(-z))


def _t_first(xp, W1, dinv):
    return pl.pallas_call(
        _t_first_body,
        grid=(G,),
        in_specs=[
            pl.BlockSpec((RB, D), lambda i: (i, 0)),
            pl.BlockSpec((H, D), lambda i: (0, 0)),
            pl.BlockSpec((RB, 1), lambda i: (i, 0)),
        ],
        out_specs=pl.BlockSpec((RB, H), lambda i: (i, 0)),
        out_shape=jax.ShapeDtypeStruct((NP, H), jnp.float32),
    )(xp, W1, dinv)


def _t_mid(agg, hs, dinv, b, W):
    return pl.pallas_call(
        _t_mid_body,
        grid=(G,),
        in_specs=[
            pl.BlockSpec((NC, RB, H), lambda i: (0, i, 0)),
            pl.BlockSpec((RB, H), lambda i: (i, 0)),
            pl.BlockSpec((RB, 1), lambda i: (i, 0)),
            pl.BlockSpec((1, H), lambda i: (0, 0)),
            pl.BlockSpec((H, H), lambda i: (0, 0)),
        ],
        out_specs=pl.BlockSpec((RB, H), lambda i: (i, 0)),
        out_shape=jax.ShapeDtypeStruct((NP, H), jnp.float32),
    )(agg, hs, dinv, b, W)


def _t_final(agg, hs, dinv, b, Wc, bc):
    return pl.pallas_call(
        _t_final_body,
        grid=(G,),
        in_specs=[
            pl.BlockSpec((NC, RB, H), lambda i: (0, i, 0)),
            pl.BlockSpec((RB, H), lambda i: (i, 0)),
            pl.BlockSpec((RB, 1), lambda i: (i, 0)),
            pl.BlockSpec((1, H), lambda i: (0, 0)),
            pl.BlockSpec((C, H), lambda i: (0, 0)),
            pl.BlockSpec((1, C), lambda i: (0, 0)),
        ],
        out_specs=pl.BlockSpec((1, C), lambda i: (0, 0)),
        out_shape=jax.ShapeDtypeStruct((1, C), jnp.float32),
        scratch_shapes=[pltpu.VMEM((1, H), jnp.float32)],
    )(agg, hs, dinv, b, Wc, bc)


def kernel(x, edge_index, W1, b1, W2, b2, W3, b3, Wc, bc):
    src = edge_index[0]
    dst = edge_index[1]
    padi = jnp.full((EPAD - E,), N, jnp.int32)
    srcp = jnp.concatenate([src, padi])
    dstp = jnp.concatenate([dst, padi])
    xp = jnp.concatenate([x, jnp.zeros((NP - N, D), x.dtype)], axis=0)

    degp = _deg_kernel(dstp)
    deg = degp[0, :, 0] + degp[1, :, 0] + 1.0
    dinv = lax.rsqrt(deg).reshape(NP, 1)

    hs1 = _t_first(xp, W1, dinv)
    agg1 = _agg_kernel(hs1, srcp, dstp)
    hs2 = _t_mid(agg1, hs1, dinv, b1.reshape(1, H), W2)
    agg2 = _agg_kernel(hs2, srcp, dstp)
    hs3 = _t_mid(agg2, hs2, dinv, b2.reshape(1, H), W3)
    agg3 = _agg_kernel(hs3, srcp, dstp)
    return _t_final(agg3, hs3, dinv, b3.reshape(1, H), Wc, bc.reshape(1, C))
